# baseline (speedup 1.0000x reference)
import functools

import jax
import jax.numpy as jnp
from jax.experimental import pallas as pl
from jax.experimental.pallas import tpu as pltpu
from jax.experimental.pallas import tpu_sc as plsc

VOCAB = 1000000
DIM = 64
BATCH = 16384
HIST = 20
NUM_IDX = BATCH * HIST

WINDOW = 128
K = 4
NROWS = NUM_IDX // WINDOW
GRID = NROWS // K

_mesh = plsc.VectorSubcoreMesh(core_axis_name="core", subcore_axis_name="subcore")

_TBLK = 2048


def _transpose_body(x_ref, o_ref):
    o_ref[...] = x_ref[...].T


@jax.jit
def _relayout_tc(table_t):
    grid = (VOCAB + _TBLK - 1) // _TBLK
    return pl.pallas_call(
        _transpose_body,
        grid=(grid,),
        in_specs=[pl.BlockSpec((DIM, _TBLK), lambda i: (0, i))],
        out_specs=pl.BlockSpec((_TBLK, DIM), lambda i: (i, 0)),
        out_shape=jax.ShapeDtypeStruct((VOCAB, DIM), table_t.dtype),
    )(table_t)


@jax.jit
def _gather(table, idx):
    @functools.partial(
        pl.kernel,
        out_type=jax.ShapeDtypeStruct((NUM_IDX, DIM), table.dtype),
        mesh=_mesh,
        scratch_types=[pltpu.SemaphoreType.DMA],
        compiler_params=pltpu.CompilerParams(use_tc_tiling_on_sc=False),
    )
    def kern(table_hbm, idx_hbm, out_hbm, sem):
        def body(i_vmem, o_vmem):
            copies = [
                pltpu.async_copy(
                    table_hbm.at[i_vmem.at[j]],
                    o_vmem.at[pl.ds(j * WINDOW, WINDOW)],
                    sem,
                )
                for j in range(K)
            ]
            for c in copies:
                c.wait()

        pltpu.emit_pipeline(
            body,
            grid=(GRID,),
            in_specs=[pl.BlockSpec((K, WINDOW), index_map=lambda i: (i, 0))],
            out_specs=[pl.BlockSpec((K * WINDOW, DIM), index_map=lambda i: (i, 0))],
            core_axis_name=("core", "subcore"),
            dimension_semantics=(pltpu.PARALLEL,),
        )(idx_hbm, out_hbm)

    return kern(table, idx)


def kernel(inputs, table):
    table_r = _relayout_tc(table.T)
    idx = inputs.reshape(NROWS, WINDOW)
    out = _gather(table_r, idx)
    return out.reshape(BATCH, HIST, DIM)


# speedup vs baseline: 1.3433x; 1.3433x over previous
import functools

import jax
import jax.numpy as jnp
from jax.experimental import pallas as pl
from jax.experimental.pallas import tpu as pltpu
from jax.experimental.pallas import tpu_sc as plsc

VOCAB = 1000000
DIM = 64
BATCH = 16384
HIST = 20

WINDOW = 128
H_BLK = 512
GRID_B = BATCH // H_BLK

_mesh = plsc.VectorSubcoreMesh(core_axis_name="core", subcore_axis_name="subcore")


@jax.jit
def _gather(table, idx_t):
    @functools.partial(
        pl.kernel,
        out_type=jax.ShapeDtypeStruct((HIST, BATCH, DIM), table.dtype),
        mesh=_mesh,
        scratch_types=[pltpu.SemaphoreType.DMA],
        compiler_params=pltpu.CompilerParams(use_tc_tiling_on_sc=False),
    )
    def kern(table_hbm, idx_hbm, out_hbm, sem):
        def body(i_vmem, o_vmem):
            copies = [
                pltpu.async_copy(
                    table_hbm.at[i_vmem.at[0, pl.ds(j * WINDOW, WINDOW)]],
                    o_vmem.at[0, pl.ds(j * WINDOW, WINDOW)],
                    sem,
                )
                for j in range(H_BLK // WINDOW)
            ]
            for c in copies:
                c.wait()

        pltpu.emit_pipeline(
            body,
            grid=(HIST, GRID_B),
            in_specs=[pl.BlockSpec((1, H_BLK), index_map=lambda h, g: (h, g))],
            out_specs=[
                pl.BlockSpec((1, H_BLK, DIM), index_map=lambda h, g: (h, g, 0))
            ],
            core_axis_name=("core", "subcore"),
            dimension_semantics=(pltpu.PARALLEL, pltpu.PARALLEL),
        )(idx_hbm, out_hbm)

    return kern(table, idx_t)


def kernel(inputs, table):
    out_h = _gather(table, inputs.T)
    return jnp.swapaxes(out_h, 0, 1)
